# Initial kernel scaffold; baseline (speedup 1.0000x reference)
import jax
import jax.numpy as jnp
from jax.experimental import pallas as pl


def kernel(z, x, params):
    raise NotImplementedError("write your pallas kernel here")


# speedup vs baseline: 2.3775x; 2.3775x over previous
import functools

import jax
import jax.numpy as jnp
from jax.experimental import pallas as pl
from jax.experimental.pallas import tpu as pltpu

K = 32


def _silu(t):
    return t * jax.nn.sigmoid(t)


def _embed_body(zf_ref, pemb_ref, temb_ref, out_ref):
    zf = zf_ref[0, 0][:, None]
    ntok = temb_ref.shape[0]
    iota = jax.lax.broadcasted_iota(jnp.int32, (zf.shape[0], ntok), 1).astype(jnp.float32)
    oh = (zf == iota).astype(jnp.float32)
    out_ref[0] = (jnp.dot(oh, temb_ref[...], preferred_element_type=jnp.float32)
                  + pemb_ref[...])


def _embed(zf, pemb, temb, T):
    B, N = zf.shape
    D = temb.shape[1]
    nj = N // T
    zf3 = zf.reshape(B * nj, 1, T)
    return pl.pallas_call(
        _embed_body,
        grid=(B, nj),
        in_specs=[
            pl.BlockSpec((1, 1, T), lambda b, j: (b * nj + j, 0, 0)),
            pl.BlockSpec((T, D), lambda b, j: (j, 0)),
            pl.BlockSpec(temb.shape, lambda b, j: (0, 0)),
        ],
        out_specs=pl.BlockSpec((1, T, D), lambda b, j: (b, j, 0)),
        out_shape=jax.ShapeDtypeStruct((B, N, D), jnp.float32),
    )(zf3, pemb, temb)


def _topk_body(x_ref, xT_ref, idx_ref, dv_ref):
    T = x_ref.shape[1]
    N = xT_ref.shape[2]
    d = None
    for c in range(3):
        col = x_ref[0, :, c][:, None]
        row = xT_ref[0, c, :][None, :]
        t = col - row
        t = t * t
        d = t if d is None else d + t
    iota = jax.lax.broadcasted_iota(jnp.int32, (T, N), 1).astype(jnp.float32)
    big = jnp.float32(3.0e38)
    cur = d
    icols, dcols = [], []
    for _ in range(K):
        mn = jnp.min(cur, axis=1, keepdims=True)
        cand = jnp.where(cur == mn, iota, jnp.float32(N))
        am = jnp.min(cand, axis=1, keepdims=True)
        icols.append(am)
        dcols.append(mn)
        cur = jnp.where(iota == am, big, cur)
    idx_ref[0] = jnp.concatenate(icols, axis=1).astype(jnp.int32)
    dv_ref[0] = jnp.concatenate(dcols, axis=1)


def _topk(coors, T):
    B, N, _ = coors.shape
    xT = jnp.swapaxes(coors, 1, 2)
    return pl.pallas_call(
        _topk_body,
        grid=(B, N // T),
        in_specs=[
            pl.BlockSpec((1, T, 3), lambda b, j: (b, j, 0)),
            pl.BlockSpec((1, 3, N), lambda b, j: (b, 0, 0)),
        ],
        out_specs=[
            pl.BlockSpec((1, T, K), lambda b, j: (b, j, 0)),
            pl.BlockSpec((1, T, K), lambda b, j: (b, j, 0)),
        ],
        out_shape=[
            jax.ShapeDtypeStruct((B, N, K), jnp.int32),
            jax.ShapeDtypeStruct((B, N, K), jnp.float32),
        ],
    )(coors, xT)


def _layer_body(feats_ref, x_ref, idx_ref, dv_ref, table_ref,
                w1a_ref, w1b_ref, wd_ref, b1_ref, w2_ref, b2_ref,
                wc1_ref, bc1_ref, wc2t_ref, bc2_ref,
                wn1a_ref, wn1b_ref, bn1_ref, wn2_ref, bn2_ref, scale_ref,
                fout_ref, cout_ref):
    T = feats_ref.shape[1]
    N = table_ref.shape[1]
    D = feats_ref.shape[2]
    E1 = w1a_ref.shape[1]
    MD = w2_ref.shape[1]
    TK = T * K

    idxf = idx_ref[0].astype(jnp.float32)
    iota = jax.lax.broadcasted_iota(jnp.int32, (TK, N), 1).astype(jnp.float32)
    oh = (idxf == iota).astype(jnp.float32)
    G = jnp.dot(oh, table_ref[0], preferred_element_type=jnp.float32)
    FJ = G[:, :D]
    XJ = G[:, D:D + 3]

    Pi = (jnp.dot(feats_ref[0], w1a_ref[...],
                  preferred_element_type=jnp.float32) + b1_ref[...])
    PJ = jnp.dot(FJ, w1b_ref[...], preferred_element_type=jnp.float32)
    dv = dv_ref[0]
    H = (PJ.reshape(T, K, E1) + Pi[:, None, :]).reshape(TK, E1)
    H = H + dv * wd_ref[...]
    H = _silu(H)
    M = jnp.dot(H, w2_ref[...], preferred_element_type=jnp.float32) + b2_ref[...]
    M = _silu(M)

    C1 = _silu(jnp.dot(M, wc1_ref[...],
                       preferred_element_type=jnp.float32) + bc1_ref[...])
    w = jnp.sum(C1 * wc2t_ref[...], axis=1, keepdims=True) + bc2_ref[...]

    nrm = jnp.sqrt(dv)
    wp = w * scale_ref[0, 0] / jnp.maximum(nrm, 1e-8)
    base = pl.program_id(1) * T
    node_id = base + jax.lax.broadcasted_iota(jnp.int32, (TK, 1), 0) // K
    wp = jnp.where(idx_ref[0] == node_id, jnp.float32(0), wp)
    rel = x_ref[0][:, None, :] - XJ.reshape(T, K, 3)
    delta = jnp.sum(wp.reshape(T, K, 1) * rel, axis=1)
    cout_ref[0] = x_ref[0] + delta

    m_i = jnp.sum(M.reshape(T, K, MD), axis=1)
    npre = (jnp.dot(feats_ref[0], wn1a_ref[...], preferred_element_type=jnp.float32)
            + jnp.dot(m_i, wn1b_ref[...], preferred_element_type=jnp.float32)
            + bn1_ref[...])
    npre = _silu(npre)
    nod = jnp.dot(npre, wn2_ref[...], preferred_element_type=jnp.float32) + bn2_ref[...]
    fout_ref[0] = feats_ref[0] + nod


def _egnn_layer(feats, coors, idx, dv, table, lp, T):
    B, N, D = feats.shape
    w1 = lp["edge1"]["w"]
    E1 = w1.shape[1]
    MD = lp["edge2"]["w"].shape[1]
    w1a, w1b, wd = w1[:D], w1[D:2 * D], w1[2 * D:2 * D + 1]
    idx = idx.reshape(B, N * K, 1)
    dv = dv.reshape(B, N * K, 1)
    args = (feats, coors, idx, dv, table,
            w1a, w1b, wd, lp["edge1"]["b"][None],
            lp["edge2"]["w"], lp["edge2"]["b"][None],
            lp["coor1"]["w"], lp["coor1"]["b"][None],
            lp["coor2"]["w"].T, lp["coor2"]["b"][None],
            lp["node1"]["w"][:D], lp["node1"]["w"][D:],
            lp["node1"]["b"][None], lp["node2"]["w"], lp["node2"]["b"][None],
            lp["coors_scale"].reshape(1, 1))
    full = lambda a: pl.BlockSpec(a.shape, lambda b, j: tuple(0 for _ in a.shape))
    in_specs = [
        pl.BlockSpec((1, T, D), lambda b, j: (b, j, 0)),
        pl.BlockSpec((1, T, 3), lambda b, j: (b, j, 0)),
        pl.BlockSpec((1, T * K, 1), lambda b, j: (b, j, 0)),
        pl.BlockSpec((1, T * K, 1), lambda b, j: (b, j, 0)),
        pl.BlockSpec((1, N, table.shape[2]), lambda b, j: (b, 0, 0)),
    ] + [full(a) for a in args[5:]]
    return pl.pallas_call(
        _layer_body,
        grid=(B, N // T),
        in_specs=in_specs,
        out_specs=[
            pl.BlockSpec((1, T, D), lambda b, j: (b, j, 0)),
            pl.BlockSpec((1, T, 3), lambda b, j: (b, j, 0)),
        ],
        out_shape=[
            jax.ShapeDtypeStruct((B, N, D), jnp.float32),
            jax.ShapeDtypeStruct((B, N, 3), jnp.float32),
        ],
    )(*args)


def _ffn_body(f_ref, g1_ref, b1n_ref, wf1_ref, bf1_ref, wf2_ref, bf2_ref,
              g2_ref, b2n_ref, out_ref):
    h = f_ref[0]
    h = h + h
    mu = jnp.mean(h, axis=1, keepdims=True)
    var = jnp.mean((h - mu) ** 2, axis=1, keepdims=True)
    h = (h - mu) / jnp.sqrt(var + 1e-5) * g1_ref[...] + b1n_ref[...]
    t = jnp.dot(h, wf1_ref[...], preferred_element_type=jnp.float32) + bf1_ref[...]
    t = jax.nn.gelu(t)
    h2 = jnp.dot(t, wf2_ref[...], preferred_element_type=jnp.float32) + bf2_ref[...]
    h = h + h2
    mu = jnp.mean(h, axis=1, keepdims=True)
    var = jnp.mean((h - mu) ** 2, axis=1, keepdims=True)
    out_ref[0] = (h - mu) / jnp.sqrt(var + 1e-5) * g2_ref[...] + b2n_ref[...]


def _ffn(feats, params, T):
    B, N, D = feats.shape
    args = (feats, params["norm1_g"][None], params["norm1_b"][None],
            params["ffn1"]["w"], params["ffn1"]["b"][None],
            params["ffn2"]["w"], params["ffn2"]["b"][None],
            params["norm2_g"][None], params["norm2_b"][None])
    full = lambda a: pl.BlockSpec(a.shape, lambda b, j: tuple(0 for _ in a.shape))
    return pl.pallas_call(
        _ffn_body,
        grid=(B, N // T),
        in_specs=[pl.BlockSpec((1, T, D), lambda b, j: (b, j, 0))]
        + [full(a) for a in args[1:]],
        out_specs=pl.BlockSpec((1, T, D), lambda b, j: (b, j, 0)),
        out_shape=jax.ShapeDtypeStruct((B, N, D), jnp.float32),
    )(*args)


def kernel(z, x, params):
    B, N, _ = x.shape
    D = params["token_emb"].shape[1]
    feats = _embed(z.astype(jnp.float32), params["pos_emb"][:N],
                   params["token_emb"], T=min(256, N))
    coors = x
    pad = jnp.zeros((B, N, 13), jnp.float32)
    for lp in params["layers"]:
        idx, dv = _topk(coors, T=16)
        table = jnp.concatenate([feats, coors, pad], axis=-1)
        feats, coors = _egnn_layer(feats, coors, idx, dv, table, lp, T=64)
    h = _ffn(feats, params, T=min(256, N))
    return h, coors


# speedup vs baseline: 5.5254x; 2.3240x over previous
import functools

import jax
import jax.numpy as jnp
from jax.experimental import pallas as pl
from jax.experimental.pallas import tpu as pltpu

K = 32


def _silu(t):
    return t * jax.nn.sigmoid(t)


def _embed_body(zf_ref, pemb_ref, temb_ref, out_ref):
    zf = zf_ref[0, 0][:, None]
    ntok = temb_ref.shape[0]
    iota = jax.lax.broadcasted_iota(jnp.int32, (zf.shape[0], ntok), 1).astype(jnp.float32)
    oh = (zf == iota).astype(jnp.float32)
    out_ref[0] = (jnp.dot(oh, temb_ref[...], preferred_element_type=jnp.float32)
                  + pemb_ref[...])


def _embed(zf, pemb, temb, T):
    B, N = zf.shape
    D = temb.shape[1]
    nj = N // T
    zf3 = zf.reshape(B * nj, 1, T)
    return pl.pallas_call(
        _embed_body,
        grid=(B, nj),
        in_specs=[
            pl.BlockSpec((1, 1, T), lambda b, j: (b * nj + j, 0, 0)),
            pl.BlockSpec((T, D), lambda b, j: (j, 0)),
            pl.BlockSpec(temb.shape, lambda b, j: (0, 0)),
        ],
        out_specs=pl.BlockSpec((1, T, D), lambda b, j: (b, j, 0)),
        out_shape=jax.ShapeDtypeStruct((B, N, D), jnp.float32),
    )(zf3, pemb, temb)


def _topk_body(x_ref, xT_ref, idx_ref, dv_ref):
    T = x_ref.shape[1]
    N = xT_ref.shape[2]
    d = None
    for c in range(3):
        col = x_ref[0, :, c][:, None]
        row = xT_ref[0, c, :][None, :]
        t = col - row
        t = t * t
        d = t if d is None else d + t
    iota = jax.lax.broadcasted_iota(jnp.int32, (T, N), 1).astype(jnp.float32)
    big = jnp.float32(3.0e38)
    cur = d
    icols, dcols = [], []
    for _ in range(K):
        mn = jnp.min(cur, axis=1, keepdims=True)
        cand = jnp.where(cur == mn, iota, jnp.float32(N))
        am = jnp.min(cand, axis=1, keepdims=True)
        icols.append(am)
        dcols.append(mn)
        cur = jnp.where(iota == am, big, cur)
    idx_ref[0] = jnp.concatenate(icols, axis=1).astype(jnp.int32)
    dv_ref[0] = jnp.concatenate(dcols, axis=1)


def _topk(coors, T):
    B, N, _ = coors.shape
    xT = jnp.swapaxes(coors, 1, 2)
    return pl.pallas_call(
        _topk_body,
        grid=(B, N // T),
        in_specs=[
            pl.BlockSpec((1, T, 3), lambda b, j: (b, j, 0)),
            pl.BlockSpec((1, 3, N), lambda b, j: (b, 0, 0)),
        ],
        out_specs=[
            pl.BlockSpec((1, T, K), lambda b, j: (b, j, 0)),
            pl.BlockSpec((1, T, K), lambda b, j: (b, j, 0)),
        ],
        out_shape=[
            jax.ShapeDtypeStruct((B, N, K), jnp.int32),
            jax.ShapeDtypeStruct((B, N, K), jnp.float32),
        ],
    )(coors, xT)


def _layer_body(feats_ref, x_ref, idx_ref, dv_ref, table_ref,
                w1a_ref, w1b_ref, wd_ref, b1_ref, w2_ref, b2_ref,
                wc1_ref, bc1_ref, wc2t_ref, bc2_ref,
                wn1a_ref, wn1b_ref, bn1_ref, wn2_ref, bn2_ref, scale_ref,
                fout_ref, cout_ref):
    T = feats_ref.shape[1]
    N = table_ref.shape[1]
    D = feats_ref.shape[2]
    E1 = w1a_ref.shape[1]
    MD = w2_ref.shape[1]
    TK = T * K

    idxf = idx_ref[0].astype(jnp.float32)
    iota = jax.lax.broadcasted_iota(jnp.int32, (TK, N), 1).astype(jnp.float32)
    oh = (idxf == iota).astype(jnp.float32)
    G = jnp.dot(oh, table_ref[0], preferred_element_type=jnp.float32)
    FJ = G[:, :D]
    XJ = G[:, D:D + 3]

    Pi = (jnp.dot(feats_ref[0], w1a_ref[...],
                  preferred_element_type=jnp.float32) + b1_ref[...])
    PJ = jnp.dot(FJ, w1b_ref[...], preferred_element_type=jnp.float32)
    dv = dv_ref[0]
    H = (PJ.reshape(T, K, E1) + Pi[:, None, :]).reshape(TK, E1)
    H = H + dv * wd_ref[...]
    H = _silu(H)
    M = jnp.dot(H, w2_ref[...], preferred_element_type=jnp.float32) + b2_ref[...]
    M = _silu(M)

    C1 = _silu(jnp.dot(M, wc1_ref[...],
                       preferred_element_type=jnp.float32) + bc1_ref[...])
    w = jnp.sum(C1 * wc2t_ref[...], axis=1, keepdims=True) + bc2_ref[...]

    nrm = jnp.sqrt(dv)
    wp = w * scale_ref[0, 0] / jnp.maximum(nrm, 1e-8)
    base = pl.program_id(1) * T
    node_id = base + jax.lax.broadcasted_iota(jnp.int32, (TK, 1), 0) // K
    wp = jnp.where(idx_ref[0] == node_id, jnp.float32(0), wp)
    rel = x_ref[0][:, None, :] - XJ.reshape(T, K, 3)
    delta = jnp.sum(wp.reshape(T, K, 1) * rel, axis=1)
    cout_ref[0] = x_ref[0] + delta

    m_i = jnp.sum(M.reshape(T, K, MD), axis=1)
    npre = (jnp.dot(feats_ref[0], wn1a_ref[...], preferred_element_type=jnp.float32)
            + jnp.dot(m_i, wn1b_ref[...], preferred_element_type=jnp.float32)
            + bn1_ref[...])
    npre = _silu(npre)
    nod = jnp.dot(npre, wn2_ref[...], preferred_element_type=jnp.float32) + bn2_ref[...]
    fout_ref[0] = feats_ref[0] + nod


def _egnn_layer(feats, coors, idx, dv, table, lp, T):
    B, N, D = feats.shape
    w1 = lp["edge1"]["w"]
    E1 = w1.shape[1]
    MD = lp["edge2"]["w"].shape[1]
    w1a, w1b, wd = w1[:D], w1[D:2 * D], w1[2 * D:2 * D + 1]
    idx = idx.reshape(B, N * K, 1)
    dv = dv.reshape(B, N * K, 1)
    args = (feats, coors, idx, dv, table,
            w1a, w1b, wd, lp["edge1"]["b"][None],
            lp["edge2"]["w"], lp["edge2"]["b"][None],
            lp["coor1"]["w"], lp["coor1"]["b"][None],
            lp["coor2"]["w"].T, lp["coor2"]["b"][None],
            lp["node1"]["w"][:D], lp["node1"]["w"][D:],
            lp["node1"]["b"][None], lp["node2"]["w"], lp["node2"]["b"][None],
            lp["coors_scale"].reshape(1, 1))
    full = lambda a: pl.BlockSpec(a.shape, lambda b, j: tuple(0 for _ in a.shape))
    in_specs = [
        pl.BlockSpec((1, T, D), lambda b, j: (b, j, 0)),
        pl.BlockSpec((1, T, 3), lambda b, j: (b, j, 0)),
        pl.BlockSpec((1, T * K, 1), lambda b, j: (b, j, 0)),
        pl.BlockSpec((1, T * K, 1), lambda b, j: (b, j, 0)),
        pl.BlockSpec((1, N, table.shape[2]), lambda b, j: (b, 0, 0)),
    ] + [full(a) for a in args[5:]]
    return pl.pallas_call(
        _layer_body,
        grid=(B, N // T),
        in_specs=in_specs,
        out_specs=[
            pl.BlockSpec((1, T, D), lambda b, j: (b, j, 0)),
            pl.BlockSpec((1, T, 3), lambda b, j: (b, j, 0)),
        ],
        out_shape=[
            jax.ShapeDtypeStruct((B, N, D), jnp.float32),
            jax.ShapeDtypeStruct((B, N, 3), jnp.float32),
        ],
    )(*args)


def _ffn_body(f_ref, g1_ref, b1n_ref, wf1_ref, bf1_ref, wf2_ref, bf2_ref,
              g2_ref, b2n_ref, out_ref):
    h = f_ref[0]
    h = h + h
    mu = jnp.mean(h, axis=1, keepdims=True)
    var = jnp.mean((h - mu) ** 2, axis=1, keepdims=True)
    h = (h - mu) / jnp.sqrt(var + 1e-5) * g1_ref[...] + b1n_ref[...]
    t = jnp.dot(h, wf1_ref[...], preferred_element_type=jnp.float32) + bf1_ref[...]
    t = jax.nn.gelu(t)
    h2 = jnp.dot(t, wf2_ref[...], preferred_element_type=jnp.float32) + bf2_ref[...]
    h = h + h2
    mu = jnp.mean(h, axis=1, keepdims=True)
    var = jnp.mean((h - mu) ** 2, axis=1, keepdims=True)
    out_ref[0] = (h - mu) / jnp.sqrt(var + 1e-5) * g2_ref[...] + b2n_ref[...]


def _ffn(feats, params, T):
    B, N, D = feats.shape
    args = (feats, params["norm1_g"][None], params["norm1_b"][None],
            params["ffn1"]["w"], params["ffn1"]["b"][None],
            params["ffn2"]["w"], params["ffn2"]["b"][None],
            params["norm2_g"][None], params["norm2_b"][None])
    full = lambda a: pl.BlockSpec(a.shape, lambda b, j: tuple(0 for _ in a.shape))
    return pl.pallas_call(
        _ffn_body,
        grid=(B, N // T),
        in_specs=[pl.BlockSpec((1, T, D), lambda b, j: (b, j, 0))]
        + [full(a) for a in args[1:]],
        out_specs=pl.BlockSpec((1, T, D), lambda b, j: (b, j, 0)),
        out_shape=jax.ShapeDtypeStruct((B, N, D), jnp.float32),
    )(*args)


def kernel(z, x, params):
    B, N, _ = x.shape
    D = params["token_emb"].shape[1]
    feats = _embed(z.astype(jnp.float32), params["pos_emb"][:N],
                   params["token_emb"], T=min(256, N))
    coors = x
    pad = jnp.zeros((B, N, 13), jnp.float32)
    for lp in params["layers"]:
        idx, dv = _topk(coors, T=256)
        table = jnp.concatenate([feats, coors, pad], axis=-1)
        feats, coors = _egnn_layer(feats, coors, idx, dv, table, lp, T=64)
    h = _ffn(feats, params, T=min(256, N))
    return h, coors
